# baseline (speedup 1.0000x reference)
import jax
import jax.numpy as jnp
from jax import lax
from jax.experimental import pallas as pl
from jax.experimental.pallas import tpu as pltpu
from jax.experimental.pallas import tpu_sc as plsc

N = 10000
NUM_GRAPHS = 64
OUT = 10
NC = 2
NS = 16
NW = NC * NS
CH = 128
ZCH = 128
NPAD = 10240
RPT = NPAD // NS
CROW = NPAD // 128


def _make_edge_agg(d, nchunk, with_cnt, nbuf):
    ept = nchunk * CH
    gb = nbuf // 2
    mesh = plsc.VectorSubcoreMesh(core_axis_name="c", subcore_axis_name="s")

    out_type = [jax.ShapeDtypeStruct((NC, NPAD, d), jnp.float32)]
    scratch = [
        pltpu.VMEM((ept,), jnp.int32),
        pltpu.VMEM((nchunk, CH), jnp.int32),
        pltpu.VMEM((nbuf, CH, d), jnp.float32),
        pltpu.VMEM((ZCH, d), jnp.float32),
        pltpu.VMEM_SHARED((NPAD, d), jnp.float32),
    ] + [pltpu.SemaphoreType.DMA] * (2 * nbuf)
    if with_cnt:
        out_type.append(jax.ShapeDtypeStruct((NC, CROW, 128), jnp.float32))
        scratch += [
            pltpu.VMEM((CROW, 128), jnp.float32),
            pltpu.VMEM((CROW,), jnp.int32),
            pltpu.VMEM_SHARED((CROW, 128), jnp.float32),
        ]

    def body(table, srcs, dsts, zeros_hbm, *rest):
        if with_cnt:
            out_p, out_c = rest[:2]
            rest = rest[2:]
        else:
            out_p = rest[0]
            rest = rest[1:]
        src_v, dst_v, rows_v, zero_v, agg_sh = rest[:5]
        gsem = rest[5:5 + nbuf]
        ssem = rest[5 + nbuf:5 + 2 * nbuf]
        if with_cnt:
            hist_v, rowidx_v, cnt_sh = rest[5 + 2 * nbuf:]

        cid = lax.axis_index("c")
        sid = lax.axis_index("s")
        wid = cid * NS + sid
        row0 = sid * RPT

        pltpu.sync_copy(srcs.at[wid], src_v)
        pltpu.sync_copy(dsts.at[wid], dst_v)
        pltpu.sync_copy(zeros_hbm, zero_v)
        for j in range(RPT // ZCH):
            pltpu.sync_copy(zero_v, agg_sh.at[pl.ds(row0 + j * ZCH, ZCH)])
        if with_cnt:
            @pl.loop(0, CROW)
            def _(rr):
                for k in range(8):
                    hist_v[rr, pl.ds(k * 16, 16)] = jnp.zeros((16,),
                                                              jnp.float32)
            for k in range(CROW // 16):
                rowidx_v[pl.ds(k * 16, 16)] = (lax.iota(jnp.int32, 16)
                                               + k * 16)
            crpt = CROW // NS
            pltpu.sync_copy(hist_v.at[pl.ds(0, crpt)],
                            cnt_sh.at[pl.ds(sid * crpt, crpt)])
        plsc.subcore_barrier()

        def start_gather(chunk, b):
            pltpu.async_copy(
                table.at[src_v.at[pl.ds(chunk * CH, CH)]], rows_v.at[b],
                gsem[b])

        def wait_gather(b):
            pltpu.make_async_copy(
                table.at[src_v.at[pl.ds(0, CH)]], rows_v.at[b],
                gsem[b]).wait()

        def start_scatter(chunk, b):
            pltpu.async_copy(rows_v.at[b], agg_sh.at[dst_v.at[chunk]],
                             ssem[b], add=True)

        def wait_scatter(b):
            pltpu.make_async_copy(rows_v.at[b], agg_sh.at[dst_v.at[0]],
                                  ssem[b]).wait()

        ngroup = nchunk // gb
        for b in range(nbuf):
            start_gather(b, b)

        @pl.loop(0, ngroup - 2, step=2)
        def _(g0):
            for p in range(2):
                cb = (g0 + p) * gb
                for k in range(gb):
                    b = p * gb + k
                    wait_gather(b)
                    start_scatter(cb + k, b)
                for k in range(gb):
                    b = p * gb + k
                    wait_scatter(b)
                    start_gather(cb + 2 * gb + k, b)

        for p in range(2):
            cb = (ngroup - 2 + p) * gb
            for k in range(gb):
                b = p * gb + k
                wait_gather(b)
                start_scatter(cb + k, b)
            for k in range(gb):
                wait_scatter(p * gb + k)

        if with_cnt:
            ones16v = jnp.ones((16,), jnp.float32)

            @pl.loop(0, nchunk)
            def _(j):
                for k in range(CH // 16):
                    dv = dst_v[j, pl.ds(k * 16, 16)]
                    plsc.addupdate_scatter(
                        hist_v, [lax.shift_right_logical(dv, 7),
                                 lax.bitwise_and(dv, 127)], ones16v)
            pltpu.sync_copy(hist_v, cnt_sh.at[rowidx_v], add=True)

        plsc.subcore_barrier()
        pltpu.sync_copy(agg_sh.at[pl.ds(row0, RPT)],
                        out_p.at[cid, pl.ds(row0, RPT)])
        if with_cnt:
            crpt = CROW // NS
            pltpu.sync_copy(cnt_sh.at[pl.ds(sid * crpt, crpt)],
                            out_c.at[cid, pl.ds(sid * crpt, crpt)])

    return pl.kernel(body, out_type=tuple(out_type), mesh=mesh,
                     scratch_types=scratch,
                     compiler_params=pltpu.CompilerParams(
                         use_tc_tiling_on_sc=False,
                         needs_layout_passes=False))


BN = 2048


def _stage_in_body(x_ref, wl_ref, wr_ref, t_ref, r_ref):
    xb = x_ref[...]
    t_ref[...] = jnp.dot(xb, wl_ref[...], preferred_element_type=jnp.float32)
    r_ref[...] = jnp.dot(xb, wr_ref[...], preferred_element_type=jnp.float32)


def _stage_in(x_pad, wlt, wrt):
    din, dout = wlt.shape
    return pl.pallas_call(
        _stage_in_body,
        grid=(NPAD // BN,),
        in_specs=[
            pl.BlockSpec((BN, din), lambda i: (i, 0)),
            pl.BlockSpec((din, dout), lambda i: (0, 0)),
            pl.BlockSpec((din, dout), lambda i: (0, 0)),
        ],
        out_specs=[
            pl.BlockSpec((BN, dout), lambda i: (i, 0)),
            pl.BlockSpec((BN, dout), lambda i: (i, 0)),
        ],
        out_shape=[
            jax.ShapeDtypeStruct((NPAD, dout), jnp.float32),
            jax.ShapeDtypeStruct((NPAD, dout), jnp.float32),
        ],
    )(x_pad, wlt, wrt)


def _stage_mid_body(p0, p1, c0, c1, r, b, wl, wr, t_ref, r_ref):
    cnt = jnp.maximum(c0[...] + c1[...], 1.0)
    h = (p0[...] + p1[...]) / cnt + b[...] + r[...]
    h = jnp.maximum(h, 0.0)
    t_ref[...] = jnp.dot(h, wl[...], preferred_element_type=jnp.float32)
    r_ref[...] = jnp.dot(h, wr[...], preferred_element_type=jnp.float32)


def _stage_mid(p0, p1, c0, c1, r, bias, wlt, wrt):
    din, dout = wlt.shape
    col = pl.BlockSpec((BN, 1), lambda i: (i, 0))
    blk = pl.BlockSpec((BN, din), lambda i: (i, 0))
    return pl.pallas_call(
        _stage_mid_body,
        grid=(NPAD // BN,),
        in_specs=[blk, blk, col, col, blk,
                  pl.BlockSpec((1, din), lambda i: (0, 0)),
                  pl.BlockSpec((din, dout), lambda i: (0, 0)),
                  pl.BlockSpec((din, dout), lambda i: (0, 0))],
        out_specs=[
            pl.BlockSpec((BN, dout), lambda i: (i, 0)),
            pl.BlockSpec((BN, dout), lambda i: (i, 0)),
        ],
        out_shape=[
            jax.ShapeDtypeStruct((NPAD, dout), jnp.float32),
            jax.ShapeDtypeStruct((NPAD, dout), jnp.float32),
        ],
    )(p0, p1, c0, c1, r, bias, wlt, wrt)


def _stage_out_body(p0, p1, c0, c1, r, bcol, b3, w1, b1, w2, b2, w3, b3b,
                    out_ref, gsum, gcnt):
    i = pl.program_id(0)

    @pl.when(i == 0)
    def _():
        gsum[...] = jnp.zeros_like(gsum)
        gcnt[...] = jnp.zeros_like(gcnt)

    cnt = jnp.maximum(c0[...] + c1[...], 1.0)
    h = (p0[...] + p1[...]) / cnt + b3[...] + r[...]
    gids = lax.broadcasted_iota(jnp.int32, (1, NUM_GRAPHS), 1)
    maskt = (bcol[...] == gids).astype(jnp.float32)
    dn = (((0,), (0,)), ((), ()))
    gsum[...] += lax.dot_general(maskt, h, dn,
                                 preferred_element_type=jnp.float32)
    ones = jnp.ones((BN, 1), jnp.float32)
    gcnt[...] += lax.dot_general(maskt, ones, dn,
                                 preferred_element_type=jnp.float32)

    @pl.when(i == pl.num_programs(0) - 1)
    def _():
        g = gsum[...] / jnp.maximum(gcnt[...], 1.0)
        z = jnp.dot(g, w1[...], preferred_element_type=jnp.float32) + b1[...]
        z = jnp.maximum(z, 0.0)
        z = jnp.dot(z, w2[...], preferred_element_type=jnp.float32) + b2[...]
        z = jnp.maximum(z, 0.0)
        out_ref[...] = (jnp.dot(z, w3[...],
                                preferred_element_type=jnp.float32) + b3b[...])


def _stage_out(p0, p1, c0, c1, r, bcol, b3, w1t, b1, w2t, b2, w3t, b3b):
    din = p0.shape[1]
    col = pl.BlockSpec((BN, 1), lambda i: (i, 0))
    blk = pl.BlockSpec((BN, din), lambda i: (i, 0))

    def full(a):
        return pl.BlockSpec(a.shape, lambda i: tuple(0 for _ in a.shape))

    return pl.pallas_call(
        _stage_out_body,
        grid=(NPAD // BN,),
        in_specs=[blk, blk, col, col, blk, col,
                  full(b3), full(w1t), full(b1), full(w2t), full(b2),
                  full(w3t), full(b3b)],
        out_specs=pl.BlockSpec((NUM_GRAPHS, OUT), lambda i: (0, 0)),
        out_shape=jax.ShapeDtypeStruct((NUM_GRAPHS, OUT), jnp.float32),
        scratch_shapes=[
            pltpu.VMEM((NUM_GRAPHS, NUM_GRAPHS), jnp.float32),
            pltpu.VMEM((NUM_GRAPHS, 1), jnp.float32),
        ],
    )(p0, p1, c0, c1, r, bcol, b3, w1t, b1, w2t, b2, w3t, b3b)


def kernel(x, edge_index, batch, W1l, b1, W1r, W2l, b2, W2r, W3l, b3, W3r,
           Wlin1, blin1, Wlin2, blin2, Wlin3, blin3):
    e = edge_index.shape[1]
    nchunk = -(-e // (NW * CH))
    nchunk = -(-nchunk // 8) * 8
    ept = nchunk * CH
    epad = NW * ept

    src = jnp.concatenate(
        [edge_index[0], jnp.zeros((epad - e,), jnp.int32)]).reshape(NW, ept)
    dst = jnp.concatenate(
        [edge_index[1],
         jnp.full((epad - e,), NPAD - 1, jnp.int32)]).reshape(NW, nchunk, CH)

    x_pad = jnp.pad(x, ((0, NPAD - N), (0, 0)))
    bcol = jnp.pad(batch, (0, NPAD - N),
                   constant_values=NUM_GRAPHS).reshape(NPAD, 1)
    zeros64 = jnp.zeros((ZCH, 64), jnp.float32)

    agg32 = _make_edge_agg(32, nchunk, True, 8)
    agg48 = _make_edge_agg(48, nchunk, False, 8)
    agg64 = _make_edge_agg(64, nchunk, False, 4)

    t1, r1 = _stage_in(x_pad, W1l.T, W1r.T)
    p1, cnt = agg32(t1, src, dst, zeros64[:, :32])
    c0 = cnt[0].reshape(NPAD, 1)
    c1 = cnt[1].reshape(NPAD, 1)
    t2, r2 = _stage_mid(p1[0], p1[1], c0, c1, r1, b1.reshape(1, 32),
                        W2l.T, W2r.T)
    (p2,) = agg48(t2, src, dst, zeros64[:, :48])
    t3, r3 = _stage_mid(p2[0], p2[1], c0, c1, r2, b2.reshape(1, 48),
                        W3l.T, W3r.T)
    (p3,) = agg64(t3, src, dst, zeros64)
    out = _stage_out(p3[0], p3[1], c0, c1, r3, bcol.astype(jnp.int32),
                     b3.reshape(1, 64), Wlin1.T, blin1.reshape(1, 32),
                     Wlin2.T, blin2.reshape(1, 32), Wlin3.T,
                     blin3.reshape(1, OUT))
    return out


# speedup vs baseline: 1.2020x; 1.2020x over previous
import jax
import jax.numpy as jnp
from jax import lax
from jax.experimental import pallas as pl
from jax.experimental.pallas import tpu as pltpu
from jax.experimental.pallas import tpu_sc as plsc

N = 10000
NUM_GRAPHS = 64
OUT = 10
NC = 2
NS = 16
NW = NC * NS
CH = 128
ZCH = 128
NPAD = 10240
RPT = NPAD // NS
CROW = NPAD // 128


def _make_edge_agg(d, nchunk, with_cnt, nbuf, bf16_in):
    ept = nchunk * CH
    gb = nbuf // 2
    mesh = plsc.VectorSubcoreMesh(core_axis_name="c", subcore_axis_name="s")

    out_type = [jax.ShapeDtypeStruct((NC, NPAD, d), jnp.float32)]
    scratch = [
        pltpu.VMEM((ept,), jnp.int32),
        pltpu.VMEM((nchunk, CH), jnp.int32),
        pltpu.VMEM((nbuf, CH, d), jnp.float32),
        pltpu.VMEM((ZCH, d), jnp.float32),
        pltpu.VMEM_SHARED((NPAD, d), jnp.float32),
    ] + [pltpu.SemaphoreType.DMA] * (2 * nbuf)
    if bf16_in:
        scratch.append(pltpu.VMEM((nbuf, CH, d), jnp.bfloat16))
    if with_cnt:
        out_type.append(jax.ShapeDtypeStruct((NC, CROW, 128), jnp.float32))
        scratch += [
            pltpu.VMEM((CROW, 128), jnp.float32),
            pltpu.VMEM((CROW,), jnp.int32),
            pltpu.VMEM_SHARED((CROW, 128), jnp.float32),
        ]

    def body(table, srcs, dsts, zeros_hbm, *rest):
        if with_cnt:
            out_p, out_c = rest[:2]
            rest = rest[2:]
        else:
            out_p = rest[0]
            rest = rest[1:]
        src_v, dst_v, rows_v, zero_v, agg_sh = rest[:5]
        rest = rest[5:]
        gsem = rest[:nbuf]
        ssem = rest[nbuf:2 * nbuf]
        rest = rest[2 * nbuf:]
        if bf16_in:
            rowsb_v = rest[0]
            rest = rest[1:]
        if with_cnt:
            hist_v, rowidx_v, cnt_sh = rest

        cid = lax.axis_index("c")
        sid = lax.axis_index("s")
        wid = cid * NS + sid
        row0 = sid * RPT

        pltpu.sync_copy(srcs.at[wid], src_v)
        pltpu.sync_copy(dsts.at[wid], dst_v)
        pltpu.sync_copy(zeros_hbm, zero_v)
        for j in range(RPT // ZCH):
            pltpu.sync_copy(zero_v, agg_sh.at[pl.ds(row0 + j * ZCH, ZCH)])
        if with_cnt:
            @pl.loop(0, CROW)
            def _(rr):
                for k in range(8):
                    hist_v[rr, pl.ds(k * 16, 16)] = jnp.zeros((16,),
                                                              jnp.float32)
            for k in range(CROW // 16):
                rowidx_v[pl.ds(k * 16, 16)] = (lax.iota(jnp.int32, 16)
                                               + k * 16)
            crpt = CROW // NS
            pltpu.sync_copy(hist_v.at[pl.ds(0, crpt)],
                            cnt_sh.at[pl.ds(sid * crpt, crpt)])
        plsc.subcore_barrier()

        grows_v = rowsb_v if bf16_in else rows_v

        def start_gather(chunk, b):
            pltpu.async_copy(
                table.at[src_v.at[pl.ds(chunk * CH, CH)]], grows_v.at[b],
                gsem[b])

        def wait_gather(b):
            pltpu.make_async_copy(
                table.at[src_v.at[pl.ds(0, CH)]], grows_v.at[b],
                gsem[b]).wait()

        def convert(b):
            if not bf16_in:
                return

            @pl.loop(0, CH)
            def _(r):
                for k in range(d // 32):
                    v = rowsb_v[b, r, pl.ds(k * 32, 32)]
                    av, bv = plsc.unpack(v,
                                         format=plsc.PackFormat.INTERLEAVED)
                    rows_v[b, r, pl.ds(k * 32, 16)] = av
                    rows_v[b, r, pl.ds(k * 32 + 16, 16)] = bv

        def start_scatter(chunk, b):
            pltpu.async_copy(rows_v.at[b], agg_sh.at[dst_v.at[chunk]],
                             ssem[b], add=True)

        def wait_scatter(b):
            pltpu.make_async_copy(rows_v.at[b], agg_sh.at[dst_v.at[0]],
                                  ssem[b]).wait()

        ngroup = nchunk // gb
        for b in range(nbuf):
            start_gather(b, b)

        @pl.loop(0, ngroup - 2, step=2)
        def _(g0):
            for p in range(2):
                cb = (g0 + p) * gb
                for k in range(gb):
                    b = p * gb + k
                    wait_gather(b)
                    convert(b)
                    start_scatter(cb + k, b)
                for k in range(gb):
                    b = p * gb + k
                    wait_scatter(b)
                    start_gather(cb + 2 * gb + k, b)

        for p in range(2):
            cb = (ngroup - 2 + p) * gb
            for k in range(gb):
                b = p * gb + k
                wait_gather(b)
                convert(b)
                start_scatter(cb + k, b)
            for k in range(gb):
                wait_scatter(p * gb + k)

        if with_cnt:
            ones16v = jnp.ones((16,), jnp.float32)

            @pl.loop(0, nchunk)
            def _(j):
                for k in range(CH // 16):
                    dv = dst_v[j, pl.ds(k * 16, 16)]
                    plsc.addupdate_scatter(
                        hist_v, [lax.shift_right_logical(dv, 7),
                                 lax.bitwise_and(dv, 127)], ones16v)
            pltpu.sync_copy(hist_v, cnt_sh.at[rowidx_v], add=True)

        plsc.subcore_barrier()
        pltpu.sync_copy(agg_sh.at[pl.ds(row0, RPT)],
                        out_p.at[cid, pl.ds(row0, RPT)])
        if with_cnt:
            crpt = CROW // NS
            pltpu.sync_copy(cnt_sh.at[pl.ds(sid * crpt, crpt)],
                            out_c.at[cid, pl.ds(sid * crpt, crpt)])

    return pl.kernel(body, out_type=tuple(out_type), mesh=mesh,
                     scratch_types=scratch,
                     compiler_params=pltpu.CompilerParams(
                         use_tc_tiling_on_sc=False,
                         needs_layout_passes=False))


BN = 2048


def _stage_in_body(x_ref, wl_ref, wr_ref, t_ref, r_ref):
    xb = x_ref[...]
    t_ref[...] = jnp.dot(xb, wl_ref[...], preferred_element_type=jnp.float32)
    r_ref[...] = jnp.dot(xb, wr_ref[...], preferred_element_type=jnp.float32)


def _stage_in(x_pad, wlt, wrt):
    din, dout = wlt.shape
    return pl.pallas_call(
        _stage_in_body,
        grid=(NPAD // BN,),
        in_specs=[
            pl.BlockSpec((BN, din), lambda i: (i, 0)),
            pl.BlockSpec((din, dout), lambda i: (0, 0)),
            pl.BlockSpec((din, dout), lambda i: (0, 0)),
        ],
        out_specs=[
            pl.BlockSpec((BN, dout), lambda i: (i, 0)),
            pl.BlockSpec((BN, dout), lambda i: (i, 0)),
        ],
        out_shape=[
            jax.ShapeDtypeStruct((NPAD, dout), jnp.float32),
            jax.ShapeDtypeStruct((NPAD, dout), jnp.float32),
        ],
    )(x_pad, wlt, wrt)


def _stage_mid_body(p0, p1, c0, c1, r, b, wl, wr, t_ref, r_ref):
    cnt = jnp.maximum(c0[...] + c1[...], 1.0)
    h = (p0[...] + p1[...]) / cnt + b[...] + r[...]
    h = jnp.maximum(h, 0.0)
    t_ref[...] = jnp.dot(h, wl[...], preferred_element_type=jnp.float32)
    r_ref[...] = jnp.dot(h, wr[...], preferred_element_type=jnp.float32)


def _stage_mid(p0, p1, c0, c1, r, bias, wlt, wrt):
    din, dout = wlt.shape
    col = pl.BlockSpec((BN, 1), lambda i: (i, 0))
    blk = pl.BlockSpec((BN, din), lambda i: (i, 0))
    return pl.pallas_call(
        _stage_mid_body,
        grid=(NPAD // BN,),
        in_specs=[blk, blk, col, col, blk,
                  pl.BlockSpec((1, din), lambda i: (0, 0)),
                  pl.BlockSpec((din, dout), lambda i: (0, 0)),
                  pl.BlockSpec((din, dout), lambda i: (0, 0))],
        out_specs=[
            pl.BlockSpec((BN, dout), lambda i: (i, 0)),
            pl.BlockSpec((BN, dout), lambda i: (i, 0)),
        ],
        out_shape=[
            jax.ShapeDtypeStruct((NPAD, dout), jnp.float32),
            jax.ShapeDtypeStruct((NPAD, dout), jnp.float32),
        ],
    )(p0, p1, c0, c1, r, bias, wlt, wrt)


def _stage_out_body(p0, p1, c0, c1, r, bcol, b3, w1, b1, w2, b2, w3, b3b,
                    out_ref, gsum, gcnt):
    i = pl.program_id(0)

    @pl.when(i == 0)
    def _():
        gsum[...] = jnp.zeros_like(gsum)
        gcnt[...] = jnp.zeros_like(gcnt)

    cnt = jnp.maximum(c0[...] + c1[...], 1.0)
    h = (p0[...] + p1[...]) / cnt + b3[...] + r[...]
    gids = lax.broadcasted_iota(jnp.int32, (1, NUM_GRAPHS), 1)
    maskt = (bcol[...] == gids).astype(jnp.float32)
    dn = (((0,), (0,)), ((), ()))
    gsum[...] += lax.dot_general(maskt, h, dn,
                                 preferred_element_type=jnp.float32)
    ones = jnp.ones((BN, 1), jnp.float32)
    gcnt[...] += lax.dot_general(maskt, ones, dn,
                                 preferred_element_type=jnp.float32)

    @pl.when(i == pl.num_programs(0) - 1)
    def _():
        g = gsum[...] / jnp.maximum(gcnt[...], 1.0)
        z = jnp.dot(g, w1[...], preferred_element_type=jnp.float32) + b1[...]
        z = jnp.maximum(z, 0.0)
        z = jnp.dot(z, w2[...], preferred_element_type=jnp.float32) + b2[...]
        z = jnp.maximum(z, 0.0)
        out_ref[...] = (jnp.dot(z, w3[...],
                                preferred_element_type=jnp.float32) + b3b[...])


def _stage_out(p0, p1, c0, c1, r, bcol, b3, w1t, b1, w2t, b2, w3t, b3b):
    din = p0.shape[1]
    col = pl.BlockSpec((BN, 1), lambda i: (i, 0))
    blk = pl.BlockSpec((BN, din), lambda i: (i, 0))

    def full(a):
        return pl.BlockSpec(a.shape, lambda i: tuple(0 for _ in a.shape))

    return pl.pallas_call(
        _stage_out_body,
        grid=(NPAD // BN,),
        in_specs=[blk, blk, col, col, blk, col,
                  full(b3), full(w1t), full(b1), full(w2t), full(b2),
                  full(w3t), full(b3b)],
        out_specs=pl.BlockSpec((NUM_GRAPHS, OUT), lambda i: (0, 0)),
        out_shape=jax.ShapeDtypeStruct((NUM_GRAPHS, OUT), jnp.float32),
        scratch_shapes=[
            pltpu.VMEM((NUM_GRAPHS, NUM_GRAPHS), jnp.float32),
            pltpu.VMEM((NUM_GRAPHS, 1), jnp.float32),
        ],
    )(p0, p1, c0, c1, r, bcol, b3, w1t, b1, w2t, b2, w3t, b3b)


def kernel(x, edge_index, batch, W1l, b1, W1r, W2l, b2, W2r, W3l, b3, W3r,
           Wlin1, blin1, Wlin2, blin2, Wlin3, blin3):
    e = edge_index.shape[1]
    nchunk = -(-e // (NW * CH))
    nchunk = -(-nchunk // 8) * 8
    ept = nchunk * CH
    epad = NW * ept

    src = jnp.concatenate(
        [edge_index[0], jnp.zeros((epad - e,), jnp.int32)]).reshape(NW, ept)
    dst = jnp.concatenate(
        [edge_index[1],
         jnp.full((epad - e,), NPAD - 1, jnp.int32)]).reshape(NW, nchunk, CH)

    x_pad = jnp.pad(x, ((0, NPAD - N), (0, 0)))
    bcol = jnp.pad(batch, (0, NPAD - N),
                   constant_values=NUM_GRAPHS).reshape(NPAD, 1)
    zeros64 = jnp.zeros((ZCH, 64), jnp.float32)

    agg32 = _make_edge_agg(32, nchunk, True, 4, True)
    agg48 = _make_edge_agg(48, nchunk, False, 4, False)
    agg64 = _make_edge_agg(64, nchunk, False, 4, True)

    def _swiz(t):
        db = t.shape[1]
        tb = t.astype(jnp.bfloat16).reshape(NPAD, db // 32, 2, 16)
        return jnp.swapaxes(tb, 2, 3).reshape(NPAD, db)

    t1, r1 = _stage_in(x_pad, W1l.T, W1r.T)
    p1, cnt = agg32(_swiz(t1), src, dst, zeros64[:, :32])
    c0 = cnt[0].reshape(NPAD, 1)
    c1 = cnt[1].reshape(NPAD, 1)
    t2, r2 = _stage_mid(p1[0], p1[1], c0, c1, r1, b1.reshape(1, 32),
                        W2l.T, W2r.T)
    (p2,) = agg48(t2, src, dst, zeros64[:, :48])
    t3, r3 = _stage_mid(p2[0], p2[1], c0, c1, r2, b2.reshape(1, 48),
                        W3l.T, W3r.T)
    (p3,) = agg64(_swiz(t3), src, dst, zeros64)
    out = _stage_out(p3[0], p3[1], c0, c1, r3, bcol.astype(jnp.int32),
                     b3.reshape(1, 64), Wlin1.T, blin1.reshape(1, 32),
                     Wlin2.T, blin2.reshape(1, 32), Wlin3.T,
                     blin3.reshape(1, OUT))
    return out


# speedup vs baseline: 1.2680x; 1.0549x over previous
import jax
import jax.numpy as jnp
from jax import lax
from jax.experimental import pallas as pl
from jax.experimental.pallas import tpu as pltpu
from jax.experimental.pallas import tpu_sc as plsc

N = 10000
NUM_GRAPHS = 64
OUT = 10
NC = 2
NS = 16
NW = NC * NS
CH = 128
ZCH = 128
NPAD = 10240
RPT = NPAD // NS
CROW = NPAD // 128


def _make_edge_agg(d, nchunk, with_cnt, nbuf, bf16_in):
    ept = nchunk * CH
    gb = nbuf // 2
    mesh = plsc.VectorSubcoreMesh(core_axis_name="c", subcore_axis_name="s")

    out_type = [jax.ShapeDtypeStruct((NC, NPAD, d), jnp.float32)]
    scratch = [
        pltpu.VMEM((ept,), jnp.int32),
        pltpu.VMEM((nchunk, CH), jnp.int32),
        pltpu.VMEM((nbuf, CH, d), jnp.float32),
        pltpu.VMEM((ZCH, d), jnp.float32),
        pltpu.VMEM_SHARED((NPAD, d), jnp.float32),
    ] + [pltpu.SemaphoreType.DMA] * (2 * nbuf)
    if bf16_in:
        scratch.append(pltpu.VMEM((nbuf, CH, d), jnp.bfloat16))
    if with_cnt:
        out_type.append(jax.ShapeDtypeStruct((NC, CROW, 128), jnp.float32))
        scratch += [
            pltpu.VMEM((CROW, 128), jnp.float32),
            pltpu.VMEM((CROW,), jnp.int32),
            pltpu.VMEM_SHARED((CROW, 128), jnp.float32),
        ]

    def body(table, srcs, dsts, zeros_hbm, *rest):
        if with_cnt:
            out_p, out_c = rest[:2]
            rest = rest[2:]
        else:
            out_p = rest[0]
            rest = rest[1:]
        src_v, dst_v, rows_v, zero_v, agg_sh = rest[:5]
        rest = rest[5:]
        gsem = rest[:nbuf]
        ssem = rest[nbuf:2 * nbuf]
        rest = rest[2 * nbuf:]
        if bf16_in:
            rowsb_v = rest[0]
            rest = rest[1:]
        if with_cnt:
            hist_v, rowidx_v, cnt_sh = rest

        cid = lax.axis_index("c")
        sid = lax.axis_index("s")
        wid = cid * NS + sid
        row0 = sid * RPT

        pltpu.sync_copy(srcs.at[wid], src_v)
        pltpu.sync_copy(dsts.at[wid], dst_v)
        pltpu.sync_copy(zeros_hbm, zero_v)
        for j in range(RPT // ZCH):
            pltpu.sync_copy(zero_v, agg_sh.at[pl.ds(row0 + j * ZCH, ZCH)])
        if with_cnt:
            @pl.loop(0, CROW)
            def _(rr):
                for k in range(8):
                    hist_v[rr, pl.ds(k * 16, 16)] = jnp.zeros((16,),
                                                              jnp.float32)
            for k in range(CROW // 16):
                rowidx_v[pl.ds(k * 16, 16)] = (lax.iota(jnp.int32, 16)
                                               + k * 16)
            crpt = CROW // NS
            pltpu.sync_copy(hist_v.at[pl.ds(0, crpt)],
                            cnt_sh.at[pl.ds(sid * crpt, crpt)])
        plsc.subcore_barrier()

        grows_v = rowsb_v if bf16_in else rows_v

        def start_gather(chunk, b):
            pltpu.async_copy(
                table.at[src_v.at[pl.ds(chunk * CH, CH)]], grows_v.at[b],
                gsem[b])

        def wait_gather(b):
            pltpu.make_async_copy(
                table.at[src_v.at[pl.ds(0, CH)]], grows_v.at[b],
                gsem[b]).wait()

        def convert(b):
            if not bf16_in:
                return

            @pl.loop(0, CH)
            def _(r):
                for k in range(d // 32):
                    v = rowsb_v[b, r, pl.ds(k * 32, 32)]
                    av, bv = plsc.unpack(v,
                                         format=plsc.PackFormat.INTERLEAVED)
                    rows_v[b, r, pl.ds(k * 32, 16)] = av
                    rows_v[b, r, pl.ds(k * 32 + 16, 16)] = bv

        def start_scatter(chunk, b):
            pltpu.async_copy(rows_v.at[b], agg_sh.at[dst_v.at[chunk]],
                             ssem[b], add=True)

        def wait_scatter(b):
            pltpu.make_async_copy(rows_v.at[b], agg_sh.at[dst_v.at[0]],
                                  ssem[b]).wait()

        ngroup = nchunk // gb
        for b in range(nbuf):
            start_gather(b, b)

        @pl.loop(0, ngroup - 2, step=2)
        def _(g0):
            for p in range(2):
                cb = (g0 + p) * gb
                for k in range(gb):
                    b = p * gb + k
                    wait_gather(b)
                    convert(b)
                    start_scatter(cb + k, b)
                for k in range(gb):
                    b = p * gb + k
                    wait_scatter(b)
                    start_gather(cb + 2 * gb + k, b)

        for p in range(2):
            cb = (ngroup - 2 + p) * gb
            for k in range(gb):
                b = p * gb + k
                wait_gather(b)
                convert(b)
                start_scatter(cb + k, b)
            for k in range(gb):
                wait_scatter(p * gb + k)

        if with_cnt:
            ones16v = jnp.ones((16,), jnp.float32)

            @pl.loop(0, nchunk)
            def _(j):
                for k in range(CH // 16):
                    dv = dst_v[j, pl.ds(k * 16, 16)]
                    plsc.addupdate_scatter(
                        hist_v, [lax.shift_right_logical(dv, 7),
                                 lax.bitwise_and(dv, 127)], ones16v)
            pltpu.sync_copy(hist_v, cnt_sh.at[rowidx_v], add=True)

        plsc.subcore_barrier()
        pltpu.sync_copy(agg_sh.at[pl.ds(row0, RPT)],
                        out_p.at[cid, pl.ds(row0, RPT)])
        if with_cnt:
            crpt = CROW // NS
            pltpu.sync_copy(cnt_sh.at[pl.ds(sid * crpt, crpt)],
                            out_c.at[cid, pl.ds(sid * crpt, crpt)])

    return pl.kernel(body, out_type=tuple(out_type), mesh=mesh,
                     scratch_types=scratch,
                     compiler_params=pltpu.CompilerParams(
                         use_tc_tiling_on_sc=False,
                         needs_layout_passes=False))


BN = 2048


def _stage_in_body(x_ref, wl_ref, wr_ref, t_ref, r_ref):
    xb = x_ref[...]
    t_ref[...] = jnp.dot(xb, wl_ref[...], preferred_element_type=jnp.float32)
    r_ref[...] = jnp.dot(xb, wr_ref[...], preferred_element_type=jnp.float32)


def _stage_in(x_pad, wlt, wrt):
    din, dout = wlt.shape
    return pl.pallas_call(
        _stage_in_body,
        grid=(NPAD // BN,),
        in_specs=[
            pl.BlockSpec((BN, din), lambda i: (i, 0)),
            pl.BlockSpec((din, dout), lambda i: (0, 0)),
            pl.BlockSpec((din, dout), lambda i: (0, 0)),
        ],
        out_specs=[
            pl.BlockSpec((BN, dout), lambda i: (i, 0)),
            pl.BlockSpec((BN, dout), lambda i: (i, 0)),
        ],
        out_shape=[
            jax.ShapeDtypeStruct((NPAD, dout), jnp.float32),
            jax.ShapeDtypeStruct((NPAD, dout), jnp.float32),
        ],
    )(x_pad, wlt, wrt)


def _stage_mid_body(p0, p1, c0, c1, r, b, wl, wr, t_ref, r_ref):
    cnt = jnp.maximum(c0[...] + c1[...], 1.0)
    h = (p0[...] + p1[...]) / cnt + b[...] + r[...]
    h = jnp.maximum(h, 0.0)
    t_ref[...] = jnp.dot(h, wl[...], preferred_element_type=jnp.float32)
    r_ref[...] = jnp.dot(h, wr[...], preferred_element_type=jnp.float32)


def _stage_mid(p0, p1, c0, c1, r, bias, wlt, wrt):
    din, dout = wlt.shape
    col = pl.BlockSpec((BN, 1), lambda i: (i, 0))
    blk = pl.BlockSpec((BN, din), lambda i: (i, 0))
    return pl.pallas_call(
        _stage_mid_body,
        grid=(NPAD // BN,),
        in_specs=[blk, blk, col, col, blk,
                  pl.BlockSpec((1, din), lambda i: (0, 0)),
                  pl.BlockSpec((din, dout), lambda i: (0, 0)),
                  pl.BlockSpec((din, dout), lambda i: (0, 0))],
        out_specs=[
            pl.BlockSpec((BN, dout), lambda i: (i, 0)),
            pl.BlockSpec((BN, dout), lambda i: (i, 0)),
        ],
        out_shape=[
            jax.ShapeDtypeStruct((NPAD, dout), jnp.float32),
            jax.ShapeDtypeStruct((NPAD, dout), jnp.float32),
        ],
    )(p0, p1, c0, c1, r, bias, wlt, wrt)


def _stage_out_body(p0, p1, c0, c1, r, bcol, b3, w1, b1, w2, b2, w3, b3b,
                    out_ref, gsum, gcnt):
    i = pl.program_id(0)

    @pl.when(i == 0)
    def _():
        gsum[...] = jnp.zeros_like(gsum)
        gcnt[...] = jnp.zeros_like(gcnt)

    cnt = jnp.maximum(c0[...] + c1[...], 1.0)
    h = (p0[...] + p1[...]) / cnt + b3[...] + r[...]
    gids = lax.broadcasted_iota(jnp.int32, (1, NUM_GRAPHS), 1)
    maskt = (bcol[...] == gids).astype(jnp.float32)
    dn = (((0,), (0,)), ((), ()))
    gsum[...] += lax.dot_general(maskt, h, dn,
                                 preferred_element_type=jnp.float32)
    ones = jnp.ones((BN, 1), jnp.float32)
    gcnt[...] += lax.dot_general(maskt, ones, dn,
                                 preferred_element_type=jnp.float32)

    @pl.when(i == pl.num_programs(0) - 1)
    def _():
        g = gsum[...] / jnp.maximum(gcnt[...], 1.0)
        z = jnp.dot(g, w1[...], preferred_element_type=jnp.float32) + b1[...]
        z = jnp.maximum(z, 0.0)
        z = jnp.dot(z, w2[...], preferred_element_type=jnp.float32) + b2[...]
        z = jnp.maximum(z, 0.0)
        out_ref[...] = (jnp.dot(z, w3[...],
                                preferred_element_type=jnp.float32) + b3b[...])


def _stage_out(p0, p1, c0, c1, r, bcol, b3, w1t, b1, w2t, b2, w3t, b3b):
    din = p0.shape[1]
    col = pl.BlockSpec((BN, 1), lambda i: (i, 0))
    blk = pl.BlockSpec((BN, din), lambda i: (i, 0))

    def full(a):
        return pl.BlockSpec(a.shape, lambda i: tuple(0 for _ in a.shape))

    return pl.pallas_call(
        _stage_out_body,
        grid=(NPAD // BN,),
        in_specs=[blk, blk, col, col, blk, col,
                  full(b3), full(w1t), full(b1), full(w2t), full(b2),
                  full(w3t), full(b3b)],
        out_specs=pl.BlockSpec((NUM_GRAPHS, OUT), lambda i: (0, 0)),
        out_shape=jax.ShapeDtypeStruct((NUM_GRAPHS, OUT), jnp.float32),
        scratch_shapes=[
            pltpu.VMEM((NUM_GRAPHS, NUM_GRAPHS), jnp.float32),
            pltpu.VMEM((NUM_GRAPHS, 1), jnp.float32),
        ],
    )(p0, p1, c0, c1, r, bcol, b3, w1t, b1, w2t, b2, w3t, b3b)


def kernel(x, edge_index, batch, W1l, b1, W1r, W2l, b2, W2r, W3l, b3, W3r,
           Wlin1, blin1, Wlin2, blin2, Wlin3, blin3):
    e = edge_index.shape[1]
    nchunk = -(-e // (NW * CH))
    nchunk = -(-nchunk // 8) * 8
    ept = nchunk * CH
    epad = NW * ept

    src = jnp.concatenate(
        [edge_index[0], jnp.zeros((epad - e,), jnp.int32)]).reshape(NW, ept)
    dst = jnp.concatenate(
        [edge_index[1],
         jnp.full((epad - e,), NPAD - 1, jnp.int32)]).reshape(NW, nchunk, CH)

    x_pad = jnp.pad(x, ((0, NPAD - N), (0, 0)))
    bcol = jnp.pad(batch, (0, NPAD - N),
                   constant_values=NUM_GRAPHS).reshape(NPAD, 1)
    zeros64 = jnp.zeros((ZCH, 64), jnp.float32)

    agg32 = _make_edge_agg(32, nchunk, True, 4, True)
    agg64 = _make_edge_agg(64, nchunk, False, 4, True)

    def _swiz(t):
        db = t.shape[1]
        tb = t.astype(jnp.bfloat16).reshape(NPAD, db // 32, 2, 16)
        return jnp.swapaxes(tb, 2, 3).reshape(NPAD, db)

    t1, r1 = _stage_in(x_pad, W1l.T, W1r.T)
    p1, cnt = agg32(_swiz(t1), src, dst, zeros64[:, :32])
    c0 = cnt[0].reshape(NPAD, 1)
    c1 = cnt[1].reshape(NPAD, 1)
    t2, r2 = _stage_mid(p1[0], p1[1], c0, c1, r1, b1.reshape(1, 32),
                        W2l.T, W2r.T)
    t2p = jnp.pad(t2, ((0, 0), (0, 16)))
    (p2,) = agg64(_swiz(t2p), src, dst, zeros64)
    t3, r3 = _stage_mid(p2[0][:, :48], p2[1][:, :48], c0, c1, r2,
                        b2.reshape(1, 48), W3l.T, W3r.T)
    (p3,) = agg64(_swiz(t3), src, dst, zeros64)
    out = _stage_out(p3[0], p3[1], c0, c1, r3, bcol.astype(jnp.int32),
                     b3.reshape(1, 64), Wlin1.T, blin1.reshape(1, 32),
                     Wlin2.T, blin2.reshape(1, 32), Wlin3.T,
                     blin3.reshape(1, OUT))
    return out
